# Initial kernel scaffold; baseline (speedup 1.0000x reference)
import jax
import jax.numpy as jnp
from jax.experimental import pallas as pl


def kernel(positions, positional_embeddings):
    raise NotImplementedError("write your pallas kernel here")


# speedup vs baseline: 1.9437x; 1.9437x over previous
import functools

import jax
import jax.numpy as jnp
from jax import lax
from jax.experimental import pallas as pl
from jax.experimental.pallas import tpu as pltpu
from jax.experimental.pallas import tpu_sc as plsc

D = 1024
NW = 32
CHUNK = 32


def _normalize_rows_body(t_ref, o_ref):
    x = t_ref[...]
    s = jnp.sum(x * x, axis=1, keepdims=True)
    norm = jnp.sqrt(s)
    o_ref[...] = x * (1.0 / jnp.maximum(norm, 1e-12))


def _normalize_table(table):
    rows, d = table.shape
    blk = 512
    return pl.pallas_call(
        _normalize_rows_body,
        grid=(rows // blk,),
        in_specs=[pl.BlockSpec((blk, d), lambda i: (i, 0))],
        out_specs=pl.BlockSpec((blk, d), lambda i: (i, 0)),
        out_shape=jax.ShapeDtypeStruct((rows, d), table.dtype),
    )(table)


def _make_sc_gather(n_rows_total):
    n_per_w = n_rows_total // NW
    n_chunks = n_per_w // CHUNK
    mesh = plsc.VectorSubcoreMesh(core_axis_name="c", subcore_axis_name="s")

    @functools.partial(
        pl.kernel,
        mesh=mesh,
        out_type=jax.ShapeDtypeStruct((n_rows_total, D), jnp.float32),
        scratch_types=[
            pltpu.VMEM((n_chunks, CHUNK), jnp.int32),
            pltpu.VMEM((CHUNK, D), jnp.float32),
            pltpu.SemaphoreType.DMA,
        ],
    )
    def gather_kernel(table_hbm, idx_hbm, out_hbm, idx_v, buf, gsem):
        wid = lax.axis_index("s") * 2 + lax.axis_index("c")
        pltpu.sync_copy(idx_hbm.at[wid], idx_v)
        base = wid * n_per_w

        def body(j, carry):
            pltpu.async_copy(table_hbm.at[idx_v.at[j]], buf, gsem).wait()
            pltpu.sync_copy(buf, out_hbm.at[pl.ds(base + j * CHUNK, CHUNK)])
            return carry

        lax.fori_loop(0, n_chunks, body, 0)

    return gather_kernel


def kernel(positions, positional_embeddings):
    b = positions.size
    n_per_w = b // NW
    n_chunks = n_per_w // CHUNK
    norm_table = _normalize_table(positional_embeddings)
    idx = positions.reshape(NW, n_chunks, CHUNK).astype(jnp.int32)
    out = _make_sc_gather(b)(norm_table, idx)
    return out.reshape(positions.shape + (D,))


# speedup vs baseline: 2.2412x; 1.1531x over previous
import functools

import jax
import jax.numpy as jnp
from jax import lax
from jax.experimental import pallas as pl
from jax.experimental.pallas import tpu as pltpu
from jax.experimental.pallas import tpu_sc as plsc

D = 1024
NW = 32
CHUNK = 16
NBUF = 4


def _normalize_rows_body(t_ref, o_ref):
    x = t_ref[...]
    s = jnp.sum(x * x, axis=1, keepdims=True)
    norm = jnp.sqrt(s)
    o_ref[...] = x * (1.0 / jnp.maximum(norm, 1e-12))


def _normalize_table(table):
    rows, d = table.shape
    blk = 512
    return pl.pallas_call(
        _normalize_rows_body,
        grid=(rows // blk,),
        in_specs=[pl.BlockSpec((blk, d), lambda i: (i, 0))],
        out_specs=pl.BlockSpec((blk, d), lambda i: (i, 0)),
        out_shape=jax.ShapeDtypeStruct((rows, d), table.dtype),
    )(table)


def _make_sc_gather(n_rows_total):
    n_per_w = n_rows_total // NW
    n_chunks = n_per_w // CHUNK
    mesh = plsc.VectorSubcoreMesh(core_axis_name="c", subcore_axis_name="s")

    @functools.partial(
        pl.kernel,
        mesh=mesh,
        out_type=jax.ShapeDtypeStruct((n_rows_total, D), jnp.float32),
        scratch_types=[
            pltpu.VMEM((n_chunks, CHUNK), jnp.int32),
            pltpu.VMEM((NBUF, CHUNK, D), jnp.float32),
            pltpu.SemaphoreType.DMA,
            pltpu.SemaphoreType.DMA,
            pltpu.SemaphoreType.DMA,
            pltpu.SemaphoreType.DMA,
            pltpu.SemaphoreType.DMA,
            pltpu.SemaphoreType.DMA,
            pltpu.SemaphoreType.DMA,
            pltpu.SemaphoreType.DMA,
        ],
    )
    def gather_kernel(table_hbm, idx_hbm, out_hbm, idx_v, buf,
                      g0, g1, g2, g3, s0, s1, s2, s3):
        gs = (g0, g1, g2, g3)
        ss = (s0, s1, s2, s3)
        wid = lax.axis_index("s") * 2 + lax.axis_index("c")
        pltpu.sync_copy(idx_hbm.at[wid], idx_v)
        base = wid * n_per_w

        def start_gather(jf, b):
            pltpu.async_copy(table_hbm.at[idx_v.at[jf]], buf.at[b], gs[b])

        def wait_gather(j, b):
            pltpu.make_async_copy(table_hbm.at[idx_v.at[j]], buf.at[b],
                                  gs[b]).wait()

        def start_scatter(j, b):
            pltpu.async_copy(buf.at[b],
                             out_hbm.at[pl.ds(base + j * CHUNK, CHUNK)], ss[b])

        def wait_scatter(j, b):
            pltpu.make_async_copy(buf.at[b],
                                  out_hbm.at[pl.ds(base + j * CHUNK, CHUNK)],
                                  ss[b]).wait()

        start_gather(0, 0)
        start_gather(1, 1)
        wait_gather(0, 0)
        start_scatter(0, 0)
        start_gather(2, 2)
        wait_gather(1, 1)
        start_scatter(1, 1)
        start_gather(3, 3)
        wait_gather(2, 2)
        start_scatter(2, 2)
        wait_scatter(0, 0)
        start_gather(4, 0)
        wait_gather(3, 3)
        start_scatter(3, 3)
        wait_scatter(1, 1)
        start_gather(5, 1)

        def body(kk, carry):
            j0 = kk * NBUF
            for b in range(NBUF):
                j = j0 + b
                bf = (b + 2) % NBUF
                wait_gather(j, b)
                start_scatter(j, b)
                wait_scatter(j - 2, bf)
                start_gather(j + 2, bf)
            return carry

        lax.fori_loop(1, n_chunks // NBUF - 1, body, 0)

        t = n_chunks - 4
        wait_gather(t + 0, 0)
        start_scatter(t + 0, 0)
        wait_scatter(t - 2, 2)
        start_gather(t + 2, 2)
        wait_gather(t + 1, 1)
        start_scatter(t + 1, 1)
        wait_scatter(t - 1, 3)
        start_gather(t + 3, 3)
        wait_gather(t + 2, 2)
        start_scatter(t + 2, 2)
        wait_gather(t + 3, 3)
        start_scatter(t + 3, 3)
        wait_scatter(t + 0, 0)
        wait_scatter(t + 1, 1)
        wait_scatter(t + 2, 2)
        wait_scatter(t + 3, 3)

    return gather_kernel


def kernel(positions, positional_embeddings):
    b = positions.size
    n_per_w = b // NW
    n_chunks = n_per_w // CHUNK
    norm_table = _normalize_table(positional_embeddings)
    idx = positions.reshape(NW, n_chunks, CHUNK).astype(jnp.int32)
    out = _make_sc_gather(b)(norm_table, idx)
    return out.reshape(positions.shape + (D,))
